# baseline (speedup 1.0000x reference)
import functools

import jax
import jax.numpy as jnp
import numpy as np
from jax import lax
from jax.experimental import pallas as pl
from jax.experimental.pallas import tpu as pltpu
from jax.experimental.pallas import tpu_sc as plsc

NUM_FACTORS = 16
CHUNK = 128


@functools.lru_cache(maxsize=None)
def _build(batch, num_users, num_items):
    info = plsc.get_sparse_core_info()
    nc, ns, lanes = info.num_cores, info.num_subcores, info.num_lanes
    nw = nc * ns
    assert lanes == NUM_FACTORS
    assert batch % (nw * CHUNK) == 0
    bpw = batch // nw
    nch = bpw // CHUNK
    ngrp = bpw // lanes

    mesh = plsc.VectorSubcoreMesh(core_axis_name="c", subcore_axis_name="s")

    @functools.partial(
        pl.kernel,
        mesh=mesh,
        compiler_params=pltpu.CompilerParams(use_tc_tiling_on_sc=False),
        out_type=jax.ShapeDtypeStruct((batch,), jnp.float32),
        scratch_types=[
            pltpu.VMEM((nch, CHUNK), jnp.int32),
            pltpu.VMEM((nch, CHUNK), jnp.int32),
            pltpu.VMEM((bpw, NUM_FACTORS), jnp.float32),
            pltpu.VMEM((bpw, NUM_FACTORS), jnp.float32),
            pltpu.VMEM((bpw,), jnp.float32),
            pltpu.VMEM((NUM_FACTORS,), jnp.float32),
            pltpu.VMEM((NUM_FACTORS,), jnp.float32),
            pltpu.SemaphoreType.DMA,
        ],
    )
    def gmf(users_hbm, items_hbm, utab_hbm, itab_hbm, w_hbm, b_hbm, out_hbm,
            uidx_v, iidx_v, ue_v, ie_v, out_v, w_v, b_v, sem):
        wid = lax.axis_index("s") * nc + lax.axis_index("c")
        base = wid * bpw

        for j in range(nch):
            pltpu.sync_copy(users_hbm.at[pl.ds(base + j * CHUNK, CHUNK)],
                            uidx_v.at[j])
            pltpu.sync_copy(items_hbm.at[pl.ds(base + j * CHUNK, CHUNK)],
                            iidx_v.at[j])
        pltpu.sync_copy(w_hbm, w_v)
        pltpu.sync_copy(b_hbm, b_v)

        copies = []
        for j in range(nch):
            copies.append(pltpu.async_copy(
                utab_hbm.at[uidx_v.at[j]],
                ue_v.at[pl.ds(j * CHUNK, CHUNK)], sem))
            copies.append(pltpu.async_copy(
                itab_hbm.at[iidx_v.at[j]],
                ie_v.at[pl.ds(j * CHUNK, CHUNK)], sem))
        for c in copies:
            c.wait()

        w_vec = w_v[...]
        b_vec = b_v[...]
        lane_iota = lax.iota(jnp.int32, lanes)
        perm = {k: lane_iota ^ k for k in (1, 2, 4, 8)}

        def lane_sum(p):
            for k in (8, 4, 2, 1):
                p = p + jax.lax.gather(
                    p,
                    perm[k][:, None],
                    jax.lax.GatherDimensionNumbers(
                        offset_dims=(), collapsed_slice_dims=(0,),
                        start_index_map=(0,)),
                    (1,),
                    mode=jax.lax.GatherScatterMode.PROMISE_IN_BOUNDS)
            return p

        def group_body(g, carry):
            base_r = g * lanes
            acc = b_vec
            for r in range(lanes):
                u = ue_v[base_r + r, :]
                i_ = ie_v[base_r + r, :]
                p = lane_sum(u * i_ * w_vec)
                acc = jnp.where(lane_iota == r, acc + p, acc)
            out_v[pl.ds(base_r, lanes)] = acc
            return carry

        lax.fori_loop(0, ngrp, group_body, 0)
        pltpu.sync_copy(out_v, out_hbm.at[pl.ds(base, bpw)])

    return gmf


def kernel(users, items, user_table, item_table, logit_w, logit_b):
    batch = users.shape[0]
    gmf = _build(batch, user_table.shape[0], item_table.shape[0])
    wvec = logit_w.reshape(NUM_FACTORS)
    bvec = jnp.broadcast_to(logit_b.reshape(()), (NUM_FACTORS,))
    return gmf(users.astype(jnp.int32), items.astype(jnp.int32),
               user_table, item_table, wvec, bvec)
